# baseline (speedup 1.0000x reference)
import functools

import jax
import jax.numpy as jnp
from jax import lax
from jax.experimental import pallas as pl
from jax.experimental.pallas import tpu as pltpu
from jax.experimental.pallas import tpu_sc as plsc

SEP_ID = 102
SEQ_LEN = 8192
EMBED_DIM = 128
NUM_CORES = 2
NUM_SUBCORES = 16
LANES = 16
NUM_WORKERS = NUM_CORES * NUM_SUBCORES
ROWS_PER_W = SEQ_LEN // NUM_WORKERS
SCAN_PER_SUB = SEQ_LEN // NUM_SUBCORES
SCAN_CHUNKS = SCAN_PER_SUB // LANES
SCAN_UNROLL = 8
NCOL = EMBED_DIM // LANES
FILL_UNROLL = 4

_mesh = plsc.VectorSubcoreMesh(core_axis_name="c", subcore_axis_name="s")


@functools.partial(
    pl.kernel,
    mesh=_mesh,
    out_type=(jax.ShapeDtypeStruct((SEQ_LEN, EMBED_DIM), jnp.float32),
              jax.ShapeDtypeStruct((NUM_WORKERS, LANES), jnp.int32)),
    scratch_types=[
        pltpu.VMEM((SCAN_PER_SUB,), jnp.int32),
        pltpu.VMEM((2, EMBED_DIM), jnp.float32),
        pltpu.VMEM((1, LANES), jnp.int32),
        pltpu.VMEM((NUM_SUBCORES, LANES), jnp.int32),
        pltpu.VMEM((3, 128, EMBED_DIM), jnp.float32),
        pltpu.VMEM_SHARED((NUM_SUBCORES, LANES), jnp.int32),
        pltpu.SemaphoreType.DMA,
    ],
)
def _seg_embed(x_hbm, table_hbm, out_hbm, stage_hbm, xv, tablev, accv,
               stagev, bufs, shared, semx):
    cid = lax.axis_index("c")
    sid = lax.axis_index("s")
    wid = sid * NUM_CORES + cid
    out_base = wid * ROWS_PER_W
    scan_base = sid * SCAN_PER_SUB

    xcopy = pltpu.async_copy(x_hbm.at[pl.ds(scan_base, SCAN_PER_SUB)], xv,
                             semx)
    pltpu.sync_copy(table_hbm, tablev)

    lane = lax.iota(jnp.int32, LANES)

    row0 = [tablev[0, pl.ds(c * LANES, LANES)] for c in range(NCOL)]
    row1 = [tablev[1, pl.ds(c * LANES, LANES)] for c in range(NCOL)]
    diff = [row1[c] - row0[c] for c in range(NCOL)]

    def runfill_body(j, _):
        for u in range(FILL_UNROLL):
            r = j * FILL_UNROLL + u
            for c in range(NCOL):
                bufs[0, r, pl.ds(c * LANES, LANES)] = row0[c]
                bufs[1, r, pl.ds(c * LANES, LANES)] = row1[c]
        return 0

    lax.fori_loop(0, 128 // FILL_UNROLL, runfill_body, 0)

    xcopy.wait()

    def scan_body(j, carry):
        acc, gidx = carry
        for u in range(SCAN_UNROLL):
            v = xv[pl.ds((j * SCAN_UNROLL + u) * LANES, LANES)]
            acc = jnp.maximum(acc, jnp.where(v == SEP_ID, gidx, -1))
            gidx = gidx + LANES
        return acc, gidx

    acc, _ = lax.fori_loop(0, SCAN_CHUNKS // SCAN_UNROLL, scan_body,
                           (jnp.full((LANES,), -1, jnp.int32),
                            scan_base + lane))
    accv[0, pl.ds(0, LANES)] = acc

    widc = cid * NUM_SUBCORES + sid
    pltpu.sync_copy(accv, stage_hbm.at[pl.ds(widc, 1)])
    plsc.subcore_barrier()

    @pl.when(cid == 0)
    def _():
        pltpu.sync_copy(stage_hbm.at[pl.ds(0, NUM_SUBCORES)], stagev)

    @pl.when(cid == 1)
    def _():
        pltpu.sync_copy(stage_hbm.at[pl.ds(NUM_SUBCORES, NUM_SUBCORES)],
                        stagev)

    mvec = stagev[0, pl.ds(0, LANES)]
    for r in range(1, NUM_SUBCORES):
        mvec = jnp.maximum(mvec, stagev[r, pl.ds(0, LANES)])

    last = mvec[0]
    for i in range(1, LANES):
        last = jnp.maximum(last, mvec[i])
    input_len = jnp.where(last >= 0, last, SEQ_LEN)

    n0 = jnp.clip(input_len - out_base, 0, ROWS_PER_W)
    zero = lane * 0

    straddle = (n0 % 128 != 0) & (n0 > 0) & (n0 < ROWS_PER_W)
    c_s = n0 // 128
    base_s = c_s * 128

    @pl.when(straddle)
    def _():
        def strad_body(j, _):
            for u in range(FILL_UNROLL):
                r = j * FILL_UNROLL + u
                m = jnp.where((zero + base_s + r) >= n0, 1, 0).astype(
                    jnp.float32)
                for c in range(NCOL):
                    bufs[2, r, pl.ds(c * LANES, LANES)] = (
                        row0[c] + m * diff[c])
            return 0

        lax.fori_loop(0, 128 // FILL_UNROLL, strad_body, 0)

    copies = []
    for c in range(2):
        lo = c * 128
        sel = jnp.where(n0 <= lo, 1, jnp.where(n0 >= lo + 128, 0, 2))
        copies.append(pltpu.async_copy(
            bufs.at[sel], out_hbm.at[pl.ds(out_base + lo, 128)], semx))
    for cp in copies:
        cp.wait()


def kernel(x, table):
    return _seg_embed(x, table)[0]


# speedup vs baseline: 1.0637x; 1.0637x over previous
import functools

import jax
import jax.numpy as jnp
from jax import lax
from jax.experimental import pallas as pl
from jax.experimental.pallas import tpu as pltpu
from jax.experimental.pallas import tpu_sc as plsc

SEP_ID = 102
SEQ_LEN = 8192
EMBED_DIM = 128
NUM_CORES = 2
NUM_SUBCORES = 16
LANES = 16
NUM_WORKERS = NUM_CORES * NUM_SUBCORES
ROWS_PER_W = SEQ_LEN // NUM_WORKERS
SCAN_PER_SUB = SEQ_LEN // NUM_SUBCORES
SCAN_CHUNKS = SCAN_PER_SUB // LANES
SCAN_UNROLL = 8
NCOL = EMBED_DIM // LANES
FILL_UNROLL = 4

_mesh = plsc.VectorSubcoreMesh(core_axis_name="c", subcore_axis_name="s")


@functools.partial(
    pl.kernel,
    mesh=_mesh,
    out_type=(jax.ShapeDtypeStruct((SEQ_LEN, EMBED_DIM), jnp.float32),
              jax.ShapeDtypeStruct((NUM_WORKERS, LANES), jnp.int32)),
    scratch_types=[
        pltpu.VMEM((SCAN_PER_SUB,), jnp.int32),
        pltpu.VMEM((2, EMBED_DIM), jnp.float32),
        pltpu.VMEM((1, LANES), jnp.int32),
        pltpu.VMEM((NUM_SUBCORES, LANES), jnp.int32),
        pltpu.VMEM((ROWS_PER_W, EMBED_DIM), jnp.float32),
        pltpu.VMEM_SHARED((NUM_SUBCORES, LANES), jnp.int32),
        pltpu.SemaphoreType.DMA,
    ],
)
def _seg_embed(x_hbm, table_hbm, out_hbm, stage_hbm, xv, tablev, accv,
               stagev, rowsv, shared, semx):
    cid = lax.axis_index("c")
    sid = lax.axis_index("s")
    wid = sid * NUM_CORES + cid
    out_base = wid * ROWS_PER_W
    scan_base = sid * SCAN_PER_SUB

    xcopy = pltpu.async_copy(x_hbm.at[pl.ds(scan_base, SCAN_PER_SUB)], xv,
                             semx)
    pltpu.sync_copy(table_hbm, tablev)

    lane = lax.iota(jnp.int32, LANES)
    xcopy.wait()

    def scan_body(j, carry):
        acc, gidx = carry
        for u in range(SCAN_UNROLL):
            v = xv[pl.ds((j * SCAN_UNROLL + u) * LANES, LANES)]
            acc = jnp.maximum(acc, jnp.where(v == SEP_ID, gidx, -1))
            gidx = gidx + LANES
        return acc, gidx

    acc, _ = lax.fori_loop(0, SCAN_CHUNKS // SCAN_UNROLL, scan_body,
                           (jnp.full((LANES,), -1, jnp.int32),
                            scan_base + lane))
    accv[0, pl.ds(0, LANES)] = acc

    widc = cid * NUM_SUBCORES + sid
    pltpu.sync_copy(accv, stage_hbm.at[pl.ds(widc, 1)])
    plsc.subcore_barrier()

    @pl.when(cid == 0)
    def _():
        pltpu.sync_copy(stage_hbm.at[pl.ds(0, NUM_SUBCORES)], stagev)

    @pl.when(cid == 1)
    def _():
        pltpu.sync_copy(stage_hbm.at[pl.ds(NUM_SUBCORES, NUM_SUBCORES)],
                        stagev)

    mvec = stagev[0, pl.ds(0, LANES)]
    for r in range(1, NUM_SUBCORES):
        mvec = jnp.maximum(mvec, stagev[r, pl.ds(0, LANES)])

    last = mvec[0]
    for i in range(1, LANES):
        last = jnp.maximum(last, mvec[i])
    input_len = jnp.where(last >= 0, last, SEQ_LEN)

    row0 = [tablev[0, pl.ds(c * LANES, LANES)] for c in range(NCOL)]
    row1 = [tablev[1, pl.ds(c * LANES, LANES)] for c in range(NCOL)]
    diff = [row1[c] - row0[c] for c in range(NCOL)]

    n0 = jnp.clip(input_len - out_base, 0, ROWS_PER_W)
    zero = lane * 0

    copies = []
    for q in range(4):
        qbase = q * 64

        def fill_body(j, _, qbase=qbase):
            for u in range(FILL_UNROLL):
                r = qbase + j * FILL_UNROLL + u
                m = jnp.where((zero + r) >= n0, 1, 0).astype(jnp.float32)
                for c in range(NCOL):
                    rowsv[r, pl.ds(c * LANES, LANES)] = (
                        row0[c] + m * diff[c])
            return 0

        lax.fori_loop(0, 64 // FILL_UNROLL, fill_body, 0)
        copies.append(pltpu.async_copy(
            rowsv.at[pl.ds(qbase, 64)],
            out_hbm.at[pl.ds(out_base + qbase, 64)], semx))
    for cp in copies:
        cp.wait()


def kernel(x, table):
    return _seg_embed(x, table)[0]
